# baseline (speedup 1.0000x reference)
import functools

import jax
import jax.numpy as jnp
from jax import lax
from jax.experimental import pallas as pl
from jax.experimental.pallas import tpu as pltpu
from jax.experimental.pallas import tpu_sc as plsc

N = 10000
E = 320000
D = 128
K = 3
NC = 2
NS = 16
EPC = E // NC
EPT = EPC // NS
CH = 128
NCH = EPT // CH
TAIL = EPT - NCH * CH
RPT = 624

def _mesh():
    return plsc.VectorSubcoreMesh(
        core_axis_name="c", subcore_axis_name="s",
        num_cores=NC, num_subcores=NS)


def _zero_fill(buf, nrows):
    zz = jnp.zeros((16,), jnp.float32)

    def zrow(i, carry):
        for k in range(D // 16):
            buf[i, pl.ds(k * 16, 16)] = zz
        return carry
    lax.fori_loop(0, nrows, zrow, 0)


def _zero_acc(buf, acc, row0, s):
    rem = RPT % CH
    for m in range(RPT // CH):
        pltpu.sync_copy(buf, acc.at[pl.ds(row0 + m * CH, CH)])
    if rem:
        pltpu.sync_copy(buf.at[pl.ds(0, rem)],
                        acc.at[pl.ds(row0 + (RPT // CH) * CH, rem)])

    @pl.when(s < 2)
    def _():
        pltpu.sync_copy(buf.at[pl.ds(0, 8)],
                        acc.at[pl.ds(NS * RPT + s * 8, 8)])


def _copy_out(acc, buf, hbm_ref, row0, c, s):
    def stage_out(r0, nrows):
        pltpu.sync_copy(acc.at[pl.ds(r0, nrows)], buf.at[pl.ds(0, nrows)])
        pltpu.sync_copy(buf.at[pl.ds(0, nrows)],
                        hbm_ref.at[c, pl.ds(r0, nrows)])

    rem = RPT % CH
    for m in range(RPT // CH):
        stage_out(row0 + m * CH, CH)
    if rem:
        stage_out(row0 + (RPT // CH) * CH, rem)

    @pl.when(s < 2)
    def _():
        stage_out(NS * RPT + s * 8, 8)


IDXW = NCH * CH
NPAIR = NCH // 2


def _stage_indices(hbm_1d, stage_1d, table_2d, base):
    pltpu.sync_copy(hbm_1d.at[pl.ds(base, IDXW)], stage_1d)

    def redis(j, carry):
        for k in range(CH // 16):
            table_2d[j, pl.ds(k * 16, 16)] = stage_1d[pl.ds(j * CH + k * 16, 16)]
        return carry
    lax.fori_loop(0, NCH, redis, 0)


def _build_agg():
    scratch = [
        pltpu.VMEM((CH,), jnp.int32),
        pltpu.VMEM((CH,), jnp.int32),
        pltpu.VMEM((CH,), jnp.int32),
        pltpu.VMEM((CH,), jnp.int32),
        pltpu.VMEM((TAIL,), jnp.int32),
        pltpu.VMEM((TAIL,), jnp.int32),
        pltpu.VMEM((CH, D), jnp.float32),
        pltpu.VMEM((CH, D), jnp.float32),
        pltpu.VMEM_SHARED((N, D), jnp.float32),
        pltpu.SemaphoreType.DMA,
        pltpu.SemaphoreType.DMA,
        pltpu.SemaphoreType.DMA,
        pltpu.SemaphoreType.DMA,
    ]

    def body(h_hbm, src_hbm, dst_hbm, out_hbm,
             srcA, dstA, srcB, dstB, srct, dstt, rowsA, rowsB, acc,
             semA, semB, semSA, semSB):
        c = lax.axis_index("c")
        s = lax.axis_index("s")
        row0 = s * RPT
        base = c * EPC + s * EPT
        _zero_fill(rowsA, CH)
        _zero_acc(rowsA, acc, row0, s)
        plsc.subcore_barrier()

        def load_idx(srcb, dstb, j):
            off = base + j * CH
            pltpu.sync_copy(src_hbm.at[pl.ds(off, CH)], srcb)
            pltpu.sync_copy(dst_hbm.at[pl.ds(off, CH)], dstb)

        load_idx(srcA, dstA, 0)
        pltpu.async_copy(h_hbm.at[srcA], rowsA, semA)
        load_idx(srcB, dstB, 1)
        pltpu.async_copy(h_hbm.at[srcB], rowsB, semB)

        def pair(g, carry):
            a = 2 * g
            b = a + 1
            pltpu.make_async_copy(h_hbm.at[srcA], rowsA, semA).wait()
            pltpu.async_copy(rowsA, acc.at[dstA], semSA, add=True)
            pltpu.make_async_copy(h_hbm.at[srcB], rowsB, semB).wait()
            pltpu.async_copy(rowsB, acc.at[dstB], semSB, add=True)
            pltpu.make_async_copy(rowsA, acc.at[dstA], semSA).wait()

            @pl.when(g < NPAIR - 1)
            def _():
                load_idx(srcA, dstA, a + 2)
                pltpu.async_copy(h_hbm.at[srcA], rowsA, semA)
            pltpu.make_async_copy(rowsB, acc.at[dstB], semSB).wait()

            @pl.when(g < NPAIR - 1)
            def _():
                load_idx(srcB, dstB, b + 2)
                pltpu.async_copy(h_hbm.at[srcB], rowsB, semB)
            return carry
        lax.fori_loop(0, NPAIR, pair, 0)

        pltpu.sync_copy(src_hbm.at[pl.ds(base + IDXW, TAIL)], srct)
        pltpu.sync_copy(dst_hbm.at[pl.ds(base + IDXW, TAIL)], dstt)
        pltpu.async_copy(h_hbm.at[srct], rowsA.at[pl.ds(0, TAIL)], semA).wait()
        pltpu.sync_copy(rowsA.at[pl.ds(0, TAIL)], acc.at[dstt], add=True)
        plsc.subcore_barrier()
        _copy_out(acc, rowsA, out_hbm, row0, c, s)

    return pl.kernel(body,
                     out_type=jax.ShapeDtypeStruct((NC, N, D), jnp.float32),
                     mesh=_mesh(), scratch_types=scratch)


def _build_deg():
    scratch = [
        pltpu.VMEM((CH, D), jnp.float32),
        pltpu.VMEM((IDXW,), jnp.int32),
        pltpu.VMEM((NCH, CH), jnp.int32),
        pltpu.VMEM((TAIL,), jnp.int32),
        pltpu.VMEM_SHARED((N, D), jnp.float32),
        pltpu.SemaphoreType.DMA,
    ]

    def body(dst_hbm, out_hbm, ones, stage1, dstall, dstt, acc, sem):
        c = lax.axis_index("c")
        s = lax.axis_index("s")
        row0 = s * RPT
        base = c * EPC + s * EPT
        _stage_indices(dst_hbm, stage1, dstall, base)
        pltpu.sync_copy(dst_hbm.at[pl.ds(base + IDXW, TAIL)], dstt)
        _zero_fill(ones, CH)
        _zero_acc(ones, acc, row0, s)

        oo = jnp.ones((16,), jnp.float32)

        def orow(i, carry):
            for k in range(D // 16):
                ones[i, pl.ds(k * 16, 16)] = oo
            return carry
        lax.fori_loop(0, CH, orow, 0)
        plsc.subcore_barrier()

        def fire(j, carry):
            pltpu.async_copy(ones, acc.at[dstall.at[j]], sem, add=True)
            return carry
        lax.fori_loop(0, NCH, fire, 0)

        def drain(j, carry):
            pltpu.make_async_copy(ones, acc.at[dstall.at[j]], sem).wait()
            return carry
        lax.fori_loop(0, NCH, drain, 0)
        pltpu.async_copy(ones.at[pl.ds(0, TAIL)], acc.at[dstt], sem,
                         add=True).wait()
        plsc.subcore_barrier()
        _copy_out(acc, ones, out_hbm, row0, c, s)

    return pl.kernel(body,
                     out_type=jax.ShapeDtypeStruct((NC, N, D), jnp.float32),
                     mesh=_mesh(), scratch_types=scratch)


_R = 2000


def _dot_t(a, w):
    return lax.dot_general(a, w, (((1,), (1,)), ((), ())),
                           preferred_element_type=jnp.float32)


def _mean_from_parts(part_ref, deg_ref):
    a = part_ref[0] + part_ref[1]
    dg = deg_ref[0, :, 0:1] + deg_ref[1, :, 0:1]
    return a * (1.0 / jnp.maximum(dg, 1.0))


def _layer_tc(part, deg16, h, Wl, bl, Wr):
    def body(part_ref, deg_ref, h_ref, wl_ref, bl_ref, wr_ref, o_ref):
        mean = _mean_from_parts(part_ref, deg_ref)
        acc = _dot_t(mean, wl_ref[...]) + _dot_t(h_ref[...], wr_ref[...])
        o_ref[...] = jnp.maximum(acc + bl_ref[...], 0.0)

    return pl.pallas_call(
        body,
        grid=(N // _R,),
        in_specs=[
            pl.BlockSpec((NC, _R, D), lambda i: (0, i, 0)),
            pl.BlockSpec((NC, _R, D), lambda i: (0, i, 0)),
            pl.BlockSpec((_R, D), lambda i: (i, 0)),
            pl.BlockSpec((D, D), lambda i: (0, 0)),
            pl.BlockSpec((1, D), lambda i: (0, 0)),
            pl.BlockSpec((D, D), lambda i: (0, 0)),
        ],
        out_specs=pl.BlockSpec((_R, D), lambda i: (i, 0)),
        out_shape=jax.ShapeDtypeStruct((N, D), jnp.float32),
    )(part, deg16, h, Wl, bl.reshape(1, D), Wr)


def _final_tc(part, deg16, h2, h1, Wl2, bl2, Wr2, W2, b2, W3, b3):
    def body(part_ref, deg_ref, h2_ref, h1_ref, wl_ref, bl_ref, wr_ref,
             w2_ref, b2_ref, w3_ref, b3_ref, o_ref):
        mean = _mean_from_parts(part_ref, deg_ref)
        h2b = h2_ref[...]
        h3 = jnp.maximum(
            _dot_t(mean, wl_ref[...]) + _dot_t(h2b, wr_ref[...]) + bl_ref[...],
            0.0)
        w2 = w2_ref[...]
        p = (_dot_t(h1_ref[...], w2[:, 0:D]) + _dot_t(h2b, w2[:, D:2 * D])
             + _dot_t(h3, w2[:, 2 * D:3 * D]) + b2_ref[...])
        p = jnp.maximum(p, 0.0)
        o_ref[...] = _dot_t(p, w3_ref[...]) + b3_ref[...]

    D2 = 2 * D
    return pl.pallas_call(
        body,
        grid=(N // _R,),
        in_specs=[
            pl.BlockSpec((NC, _R, D), lambda i: (0, i, 0)),
            pl.BlockSpec((NC, _R, D), lambda i: (0, i, 0)),
            pl.BlockSpec((_R, D), lambda i: (i, 0)),
            pl.BlockSpec((_R, D), lambda i: (i, 0)),
            pl.BlockSpec((D, D), lambda i: (0, 0)),
            pl.BlockSpec((1, D), lambda i: (0, 0)),
            pl.BlockSpec((D, D), lambda i: (0, 0)),
            pl.BlockSpec((D2, K * D), lambda i: (0, 0)),
            pl.BlockSpec((1, D2), lambda i: (0, 0)),
            pl.BlockSpec((D, D2), lambda i: (0, 0)),
            pl.BlockSpec((1, D), lambda i: (0, 0)),
        ],
        out_specs=pl.BlockSpec((_R, D), lambda i: (i, 0)),
        out_shape=jax.ShapeDtypeStruct((N, D), jnp.float32),
    )(part, deg16, h2, h1, Wl2, bl2.reshape(1, D), Wr2,
      W2, b2.reshape(1, D2), W3, b3.reshape(1, D))


def kernel(x, edge_index, Wl0, bl0, Wr0, Wl1, bl1, Wr1, Wl2, bl2, Wr2,
           W2, b2, W3, b3):
    src = edge_index[0].astype(jnp.int32)
    dst = edge_index[1].astype(jnp.int32)
    agg = _build_agg()
    degp = _build_deg()(dst)
    part0 = agg(x, src, dst)
    h1 = _layer_tc(part0, degp, x, Wl0, bl0, Wr0)
    part1 = agg(h1, src, dst)
    h2 = _layer_tc(part1, degp, h1, Wl1, bl1, Wr1)
    part2 = agg(h2, src, dst)
    return _final_tc(part2, degp, h2, h1, Wl2, bl2, Wr2, W2, b2, W3, b3)


# speedup vs baseline: 1.0833x; 1.0833x over previous
import functools

import jax
import jax.numpy as jnp
from jax import lax
from jax.experimental import pallas as pl
from jax.experimental.pallas import tpu as pltpu
from jax.experimental.pallas import tpu_sc as plsc

N = 10000
E = 320000
D = 128
K = 3
NC = 2
NS = 16
EPC = E // NC
EPT = EPC // NS
CH = 128
NCH = EPT // CH
TAIL = EPT - NCH * CH
RPT = 624

def _mesh():
    return plsc.VectorSubcoreMesh(
        core_axis_name="c", subcore_axis_name="s",
        num_cores=NC, num_subcores=NS)


def _zero_fill(buf, nrows):
    zz = jnp.zeros((16,), jnp.float32)

    def zrow(i, carry):
        for k in range(D // 16):
            buf[i, pl.ds(k * 16, 16)] = zz
        return carry
    lax.fori_loop(0, nrows, zrow, 0)


def _zero_acc(buf, acc, row0, s):
    rem = RPT % CH
    for m in range(RPT // CH):
        pltpu.sync_copy(buf, acc.at[pl.ds(row0 + m * CH, CH)])
    if rem:
        pltpu.sync_copy(buf.at[pl.ds(0, rem)],
                        acc.at[pl.ds(row0 + (RPT // CH) * CH, rem)])

    @pl.when(s < 2)
    def _():
        pltpu.sync_copy(buf.at[pl.ds(0, 8)],
                        acc.at[pl.ds(NS * RPT + s * 8, 8)])


def _copy_out(acc, buf, hbm_ref, row0, c, s):
    def stage_out(r0, nrows):
        pltpu.sync_copy(acc.at[pl.ds(r0, nrows)], buf.at[pl.ds(0, nrows)])
        pltpu.sync_copy(buf.at[pl.ds(0, nrows)],
                        hbm_ref.at[c, pl.ds(r0, nrows)])

    rem = RPT % CH
    for m in range(RPT // CH):
        stage_out(row0 + m * CH, CH)
    if rem:
        stage_out(row0 + (RPT // CH) * CH, rem)

    @pl.when(s < 2)
    def _():
        stage_out(NS * RPT + s * 8, 8)


IDXW = NCH * CH
NPAIR = NCH // 2


def _stage_indices(hbm_1d, stage_1d, table_2d, base):
    pltpu.sync_copy(hbm_1d.at[pl.ds(base, IDXW)], stage_1d)

    def redis(j, carry):
        for k in range(CH // 16):
            table_2d[j, pl.ds(k * 16, 16)] = stage_1d[pl.ds(j * CH + k * 16, 16)]
        return carry
    lax.fori_loop(0, NCH, redis, 0)


def _build_agg():
    scratch = [
        pltpu.VMEM((CH,), jnp.int32),
        pltpu.VMEM((CH,), jnp.int32),
        pltpu.VMEM((CH,), jnp.int32),
        pltpu.VMEM((CH,), jnp.int32),
        pltpu.VMEM((TAIL,), jnp.int32),
        pltpu.VMEM((TAIL,), jnp.int32),
        pltpu.VMEM((CH, D), jnp.float32),
        pltpu.VMEM((CH, D), jnp.float32),
        pltpu.VMEM_SHARED((N, D), jnp.float32),
        pltpu.SemaphoreType.DMA,
        pltpu.SemaphoreType.DMA,
        pltpu.SemaphoreType.DMA,
        pltpu.SemaphoreType.DMA,
    ]

    def body(h_hbm, src_hbm, dst_hbm, out_hbm,
             srcA, dstA, srcB, dstB, srct, dstt, rowsA, rowsB, acc,
             semA, semB, semIA, semIB):
        c = lax.axis_index("c")
        s = lax.axis_index("s")
        row0 = s * RPT
        base = c * EPC + s * EPT
        _zero_fill(rowsA, CH)
        _zero_acc(rowsA, acc, row0, s)
        plsc.subcore_barrier()

        def fire_idx(srcb, dstb, j, semI):
            off = base + j * CH
            pltpu.async_copy(src_hbm.at[pl.ds(off, CH)], srcb, semI)
            pltpu.async_copy(dst_hbm.at[pl.ds(off, CH)], dstb, semI)

        def wait_idx(srcb, dstb, j, semI):
            off = base + j * CH
            pltpu.make_async_copy(src_hbm.at[pl.ds(off, CH)], srcb,
                                  semI).wait()
            pltpu.make_async_copy(dst_hbm.at[pl.ds(off, CH)], dstb,
                                  semI).wait()

        fire_idx(srcA, dstA, 0, semIA)
        wait_idx(srcA, dstA, 0, semIA)
        pltpu.async_copy(h_hbm.at[srcA], rowsA, semA)
        fire_idx(srcB, dstB, 1, semIB)

        def pair(g, carry):
            a = 2 * g
            b = a + 1
            wait_idx(srcB, dstB, b, semIB)
            pltpu.async_copy(h_hbm.at[srcB], rowsB, semB)
            pltpu.make_async_copy(h_hbm.at[srcA], rowsA, semA).wait()
            pltpu.sync_copy(rowsA, acc.at[dstA], add=True)

            @pl.when(g < NPAIR - 1)
            def _():
                fire_idx(srcA, dstA, a + 2, semIA)
            pltpu.make_async_copy(h_hbm.at[srcB], rowsB, semB).wait()
            pltpu.sync_copy(rowsB, acc.at[dstB], add=True)

            @pl.when(g < NPAIR - 1)
            def _():
                wait_idx(srcA, dstA, a + 2, semIA)
                pltpu.async_copy(h_hbm.at[srcA], rowsA, semA)
                fire_idx(srcB, dstB, b + 2, semIB)
            return carry
        lax.fori_loop(0, NPAIR, pair, 0)

        pltpu.sync_copy(src_hbm.at[pl.ds(base + IDXW, TAIL)], srct)
        pltpu.sync_copy(dst_hbm.at[pl.ds(base + IDXW, TAIL)], dstt)
        pltpu.async_copy(h_hbm.at[srct], rowsA.at[pl.ds(0, TAIL)], semA).wait()
        pltpu.sync_copy(rowsA.at[pl.ds(0, TAIL)], acc.at[dstt], add=True)
        plsc.subcore_barrier()
        _copy_out(acc, rowsA, out_hbm, row0, c, s)

    return pl.kernel(body,
                     out_type=jax.ShapeDtypeStruct((NC, N, D), jnp.float32),
                     mesh=_mesh(), scratch_types=scratch)


def _build_deg():
    scratch = [
        pltpu.VMEM((CH, D), jnp.float32),
        pltpu.VMEM((IDXW,), jnp.int32),
        pltpu.VMEM((NCH, CH), jnp.int32),
        pltpu.VMEM((TAIL,), jnp.int32),
        pltpu.VMEM_SHARED((N, D), jnp.float32),
        pltpu.SemaphoreType.DMA,
    ]

    def body(dst_hbm, out_hbm, ones, stage1, dstall, dstt, acc, sem):
        c = lax.axis_index("c")
        s = lax.axis_index("s")
        row0 = s * RPT
        base = c * EPC + s * EPT
        _stage_indices(dst_hbm, stage1, dstall, base)
        pltpu.sync_copy(dst_hbm.at[pl.ds(base + IDXW, TAIL)], dstt)
        _zero_fill(ones, CH)
        _zero_acc(ones, acc, row0, s)

        oo = jnp.ones((16,), jnp.float32)

        def orow(i, carry):
            for k in range(D // 16):
                ones[i, pl.ds(k * 16, 16)] = oo
            return carry
        lax.fori_loop(0, CH, orow, 0)
        plsc.subcore_barrier()

        def fire(j, carry):
            pltpu.async_copy(ones, acc.at[dstall.at[j]], sem, add=True)
            return carry
        lax.fori_loop(0, NCH, fire, 0)

        def drain(j, carry):
            pltpu.make_async_copy(ones, acc.at[dstall.at[j]], sem).wait()
            return carry
        lax.fori_loop(0, NCH, drain, 0)
        pltpu.async_copy(ones.at[pl.ds(0, TAIL)], acc.at[dstt], sem,
                         add=True).wait()
        plsc.subcore_barrier()
        _copy_out(acc, ones, out_hbm, row0, c, s)

    return pl.kernel(body,
                     out_type=jax.ShapeDtypeStruct((NC, N, D), jnp.float32),
                     mesh=_mesh(), scratch_types=scratch)


_R = 2000


def _dot_t(a, w):
    return lax.dot_general(a, w, (((1,), (1,)), ((), ())),
                           preferred_element_type=jnp.float32)


def _mean_from_parts(part_ref, deg_ref):
    a = part_ref[0] + part_ref[1]
    dg = deg_ref[0, :, 0:1] + deg_ref[1, :, 0:1]
    return a * (1.0 / jnp.maximum(dg, 1.0))


def _layer_tc(part, deg16, h, Wl, bl, Wr):
    def body(part_ref, deg_ref, h_ref, wl_ref, bl_ref, wr_ref, o_ref):
        mean = _mean_from_parts(part_ref, deg_ref)
        acc = _dot_t(mean, wl_ref[...]) + _dot_t(h_ref[...], wr_ref[...])
        o_ref[...] = jnp.maximum(acc + bl_ref[...], 0.0)

    return pl.pallas_call(
        body,
        grid=(N // _R,),
        in_specs=[
            pl.BlockSpec((NC, _R, D), lambda i: (0, i, 0)),
            pl.BlockSpec((NC, _R, D), lambda i: (0, i, 0)),
            pl.BlockSpec((_R, D), lambda i: (i, 0)),
            pl.BlockSpec((D, D), lambda i: (0, 0)),
            pl.BlockSpec((1, D), lambda i: (0, 0)),
            pl.BlockSpec((D, D), lambda i: (0, 0)),
        ],
        out_specs=pl.BlockSpec((_R, D), lambda i: (i, 0)),
        out_shape=jax.ShapeDtypeStruct((N, D), jnp.float32),
    )(part, deg16, h, Wl, bl.reshape(1, D), Wr)


def _final_tc(part, deg16, h2, h1, Wl2, bl2, Wr2, W2, b2, W3, b3):
    def body(part_ref, deg_ref, h2_ref, h1_ref, wl_ref, bl_ref, wr_ref,
             w2_ref, b2_ref, w3_ref, b3_ref, o_ref):
        mean = _mean_from_parts(part_ref, deg_ref)
        h2b = h2_ref[...]
        h3 = jnp.maximum(
            _dot_t(mean, wl_ref[...]) + _dot_t(h2b, wr_ref[...]) + bl_ref[...],
            0.0)
        w2 = w2_ref[...]
        p = (_dot_t(h1_ref[...], w2[:, 0:D]) + _dot_t(h2b, w2[:, D:2 * D])
             + _dot_t(h3, w2[:, 2 * D:3 * D]) + b2_ref[...])
        p = jnp.maximum(p, 0.0)
        o_ref[...] = _dot_t(p, w3_ref[...]) + b3_ref[...]

    D2 = 2 * D
    return pl.pallas_call(
        body,
        grid=(N // _R,),
        in_specs=[
            pl.BlockSpec((NC, _R, D), lambda i: (0, i, 0)),
            pl.BlockSpec((NC, _R, D), lambda i: (0, i, 0)),
            pl.BlockSpec((_R, D), lambda i: (i, 0)),
            pl.BlockSpec((_R, D), lambda i: (i, 0)),
            pl.BlockSpec((D, D), lambda i: (0, 0)),
            pl.BlockSpec((1, D), lambda i: (0, 0)),
            pl.BlockSpec((D, D), lambda i: (0, 0)),
            pl.BlockSpec((D2, K * D), lambda i: (0, 0)),
            pl.BlockSpec((1, D2), lambda i: (0, 0)),
            pl.BlockSpec((D, D2), lambda i: (0, 0)),
            pl.BlockSpec((1, D), lambda i: (0, 0)),
        ],
        out_specs=pl.BlockSpec((_R, D), lambda i: (i, 0)),
        out_shape=jax.ShapeDtypeStruct((N, D), jnp.float32),
    )(part, deg16, h2, h1, Wl2, bl2.reshape(1, D), Wr2,
      W2, b2.reshape(1, D2), W3, b3.reshape(1, D))


def kernel(x, edge_index, Wl0, bl0, Wr0, Wl1, bl1, Wr1, Wl2, bl2, Wr2,
           W2, b2, W3, b3):
    src = edge_index[0].astype(jnp.int32)
    dst = edge_index[1].astype(jnp.int32)
    agg = _build_agg()
    degp = _build_deg()(dst)
    part0 = agg(x, src, dst)
    h1 = _layer_tc(part0, degp, x, Wl0, bl0, Wr0)
    part1 = agg(h1, src, dst)
    h2 = _layer_tc(part1, degp, h1, Wl1, bl1, Wr1)
    part2 = agg(h2, src, dst)
    return _final_tc(part2, degp, h2, h1, Wl2, bl2, Wr2, W2, b2, W3, b3)


# speedup vs baseline: 1.0966x; 1.0123x over previous
import functools

import jax
import jax.numpy as jnp
from jax import lax
from jax.experimental import pallas as pl
from jax.experimental.pallas import tpu as pltpu
from jax.experimental.pallas import tpu_sc as plsc

N = 10000
E = 320000
D = 128
K = 3
NC = 2
NS = 16
EPC = E // NC
EPT = EPC // NS
CH = 128
NCH = EPT // CH
TAIL = EPT - NCH * CH
RPT = 624

def _mesh():
    return plsc.VectorSubcoreMesh(
        core_axis_name="c", subcore_axis_name="s",
        num_cores=NC, num_subcores=NS)


def _zero_fill(buf, nrows):
    zz = jnp.zeros((16,), jnp.float32)

    def zrow(i, carry):
        for k in range(D // 16):
            buf[i, pl.ds(k * 16, 16)] = zz
        return carry
    lax.fori_loop(0, nrows, zrow, 0)


def _acc_slices(row0):
    rem = RPT % CH
    slices = [(row0 + m * CH, CH) for m in range(RPT // CH)]
    if rem:
        slices.append((row0 + (RPT // CH) * CH, rem))
    return slices


def _zero_acc(buf, acc, row0, s, sem):
    for r0, nr in _acc_slices(row0):
        pltpu.async_copy(buf.at[pl.ds(0, nr)], acc.at[pl.ds(r0, nr)], sem)

    @pl.when(s < 2)
    def _():
        pltpu.async_copy(buf.at[pl.ds(0, 8)],
                         acc.at[pl.ds(NS * RPT + s * 8, 8)], sem)
    for r0, nr in _acc_slices(row0):
        pltpu.make_async_copy(buf.at[pl.ds(0, nr)],
                              acc.at[pl.ds(r0, nr)], sem).wait()

    @pl.when(s < 2)
    def _():
        pltpu.make_async_copy(buf.at[pl.ds(0, 8)],
                              acc.at[pl.ds(NS * RPT + s * 8, 8)], sem).wait()


def _copy_out(acc, buf, hbm_ref, row0, c, s):
    def stage_out(r0, nrows):
        pltpu.sync_copy(acc.at[pl.ds(r0, nrows)], buf.at[pl.ds(0, nrows)])
        pltpu.sync_copy(buf.at[pl.ds(0, nrows)],
                        hbm_ref.at[c, pl.ds(r0, nrows)])

    for r0, nr in _acc_slices(row0):
        stage_out(r0, nr)

    @pl.when(s < 2)
    def _():
        stage_out(NS * RPT + s * 8, 8)


def _copy_out2(acc, bufs, hbm_ref, row0, c, s, semH):
    stages = _acc_slices(row0)

    def b2h(m):
        r0, nr = stages[m]
        return pltpu.make_async_copy(bufs[m % 2].at[pl.ds(0, nr)],
                                     hbm_ref.at[c, pl.ds(r0, nr)],
                                     semH[m % 2])

    for m, (r0, nr) in enumerate(stages):
        i = m % 2
        if m >= 2:
            b2h(m - 2).wait()
        pltpu.sync_copy(acc.at[pl.ds(r0, nr)], bufs[i].at[pl.ds(0, nr)])
        b2h(m).start()
    for m in range(max(0, len(stages) - 2), len(stages)):
        b2h(m).wait()

    @pl.when(s < 2)
    def _():
        r0 = NS * RPT + s * 8
        pltpu.sync_copy(acc.at[pl.ds(r0, 8)], bufs[0].at[pl.ds(0, 8)])
        pltpu.sync_copy(bufs[0].at[pl.ds(0, 8)], hbm_ref.at[c, pl.ds(r0, 8)])


IDXW = NCH * CH
NPAIR = NCH // 2


def _stage_indices(hbm_1d, stage_1d, table_2d, base):
    pltpu.sync_copy(hbm_1d.at[pl.ds(base, IDXW)], stage_1d)

    def redis(j, carry):
        for k in range(CH // 16):
            table_2d[j, pl.ds(k * 16, 16)] = stage_1d[pl.ds(j * CH + k * 16, 16)]
        return carry
    lax.fori_loop(0, NCH, redis, 0)


def _build_agg():
    scratch = [
        pltpu.VMEM((CH,), jnp.int32),
        pltpu.VMEM((CH,), jnp.int32),
        pltpu.VMEM((CH,), jnp.int32),
        pltpu.VMEM((CH,), jnp.int32),
        pltpu.VMEM((TAIL,), jnp.int32),
        pltpu.VMEM((TAIL,), jnp.int32),
        pltpu.VMEM((CH, D), jnp.float32),
        pltpu.VMEM((CH, D), jnp.float32),
        pltpu.VMEM_SHARED((N, D), jnp.float32),
        pltpu.SemaphoreType.DMA,
        pltpu.SemaphoreType.DMA,
        pltpu.SemaphoreType.DMA,
        pltpu.SemaphoreType.DMA,
    ]

    def body(h_hbm, src_hbm, dst_hbm, out_hbm,
             srcA, dstA, srcB, dstB, srct, dstt, rowsA, rowsB, acc,
             semA, semB, semIA, semIB):
        c = lax.axis_index("c")
        s = lax.axis_index("s")
        row0 = s * RPT
        base = c * EPC + s * EPT
        _zero_fill(rowsA, CH)
        _zero_acc(rowsA, acc, row0, s, semA)
        plsc.subcore_barrier()

        def fire_idx(srcb, dstb, j, semI):
            off = base + j * CH
            pltpu.async_copy(src_hbm.at[pl.ds(off, CH)], srcb, semI)
            pltpu.async_copy(dst_hbm.at[pl.ds(off, CH)], dstb, semI)

        def wait_idx(srcb, dstb, j, semI):
            off = base + j * CH
            pltpu.make_async_copy(src_hbm.at[pl.ds(off, CH)], srcb,
                                  semI).wait()
            pltpu.make_async_copy(dst_hbm.at[pl.ds(off, CH)], dstb,
                                  semI).wait()

        fire_idx(srcA, dstA, 0, semIA)
        wait_idx(srcA, dstA, 0, semIA)
        pltpu.async_copy(h_hbm.at[srcA], rowsA, semA)
        fire_idx(srcB, dstB, 1, semIB)

        def pair(g, carry):
            a = 2 * g
            b = a + 1
            wait_idx(srcB, dstB, b, semIB)
            pltpu.async_copy(h_hbm.at[srcB], rowsB, semB)
            pltpu.make_async_copy(h_hbm.at[srcA], rowsA, semA).wait()
            pltpu.sync_copy(rowsA, acc.at[dstA], add=True)

            @pl.when(g < NPAIR - 1)
            def _():
                fire_idx(srcA, dstA, a + 2, semIA)
            pltpu.make_async_copy(h_hbm.at[srcB], rowsB, semB).wait()
            pltpu.sync_copy(rowsB, acc.at[dstB], add=True)

            @pl.when(g < NPAIR - 1)
            def _():
                wait_idx(srcA, dstA, a + 2, semIA)
                pltpu.async_copy(h_hbm.at[srcA], rowsA, semA)
                fire_idx(srcB, dstB, b + 2, semIB)
            return carry
        lax.fori_loop(0, NPAIR, pair, 0)

        pltpu.sync_copy(src_hbm.at[pl.ds(base + IDXW, TAIL)], srct)
        pltpu.sync_copy(dst_hbm.at[pl.ds(base + IDXW, TAIL)], dstt)
        pltpu.async_copy(h_hbm.at[srct], rowsA.at[pl.ds(0, TAIL)], semA).wait()
        pltpu.sync_copy(rowsA.at[pl.ds(0, TAIL)], acc.at[dstt], add=True)
        plsc.subcore_barrier()
        _copy_out2(acc, (rowsA, rowsB), out_hbm, row0, c, s,
                   (semA, semB))

    return pl.kernel(body,
                     out_type=jax.ShapeDtypeStruct((NC, N, D), jnp.float32),
                     mesh=_mesh(), scratch_types=scratch)


def _build_deg():
    scratch = [
        pltpu.VMEM((CH, D), jnp.float32),
        pltpu.VMEM((IDXW,), jnp.int32),
        pltpu.VMEM((NCH, CH), jnp.int32),
        pltpu.VMEM((TAIL,), jnp.int32),
        pltpu.VMEM_SHARED((N, D), jnp.float32),
        pltpu.SemaphoreType.DMA,
    ]

    def body(dst_hbm, out_hbm, ones, stage1, dstall, dstt, acc, sem):
        c = lax.axis_index("c")
        s = lax.axis_index("s")
        row0 = s * RPT
        base = c * EPC + s * EPT
        _stage_indices(dst_hbm, stage1, dstall, base)
        pltpu.sync_copy(dst_hbm.at[pl.ds(base + IDXW, TAIL)], dstt)
        _zero_fill(ones, CH)
        _zero_acc(ones, acc, row0, s, sem)

        oo = jnp.ones((16,), jnp.float32)

        def orow(i, carry):
            for k in range(D // 16):
                ones[i, pl.ds(k * 16, 16)] = oo
            return carry
        lax.fori_loop(0, CH, orow, 0)
        plsc.subcore_barrier()

        def fire(j, carry):
            pltpu.async_copy(ones, acc.at[dstall.at[j]], sem, add=True)
            return carry
        lax.fori_loop(0, NCH, fire, 0)

        def drain(j, carry):
            pltpu.make_async_copy(ones, acc.at[dstall.at[j]], sem).wait()
            return carry
        lax.fori_loop(0, NCH, drain, 0)
        pltpu.async_copy(ones.at[pl.ds(0, TAIL)], acc.at[dstt], sem,
                         add=True).wait()
        plsc.subcore_barrier()
        _copy_out(acc, ones, out_hbm, row0, c, s)

    return pl.kernel(body,
                     out_type=jax.ShapeDtypeStruct((NC, N, D), jnp.float32),
                     mesh=_mesh(), scratch_types=scratch)


_R = 2000


def _dot_t(a, w):
    return lax.dot_general(a, w, (((1,), (1,)), ((), ())),
                           preferred_element_type=jnp.float32)


def _mean_from_parts(part_ref, deg_ref):
    a = part_ref[0] + part_ref[1]
    dg = deg_ref[0, :, 0:1] + deg_ref[1, :, 0:1]
    return a * (1.0 / jnp.maximum(dg, 1.0))


def _layer_tc(part, deg16, h, Wl, bl, Wr):
    def body(part_ref, deg_ref, h_ref, wl_ref, bl_ref, wr_ref, o_ref):
        mean = _mean_from_parts(part_ref, deg_ref)
        acc = _dot_t(mean, wl_ref[...]) + _dot_t(h_ref[...], wr_ref[...])
        o_ref[...] = jnp.maximum(acc + bl_ref[...], 0.0)

    return pl.pallas_call(
        body,
        grid=(N // _R,),
        in_specs=[
            pl.BlockSpec((NC, _R, D), lambda i: (0, i, 0)),
            pl.BlockSpec((NC, _R, D), lambda i: (0, i, 0)),
            pl.BlockSpec((_R, D), lambda i: (i, 0)),
            pl.BlockSpec((D, D), lambda i: (0, 0)),
            pl.BlockSpec((1, D), lambda i: (0, 0)),
            pl.BlockSpec((D, D), lambda i: (0, 0)),
        ],
        out_specs=pl.BlockSpec((_R, D), lambda i: (i, 0)),
        out_shape=jax.ShapeDtypeStruct((N, D), jnp.float32),
    )(part, deg16, h, Wl, bl.reshape(1, D), Wr)


def _final_tc(part, deg16, h2, h1, Wl2, bl2, Wr2, W2, b2, W3, b3):
    def body(part_ref, deg_ref, h2_ref, h1_ref, wl_ref, bl_ref, wr_ref,
             w2_ref, b2_ref, w3_ref, b3_ref, o_ref):
        mean = _mean_from_parts(part_ref, deg_ref)
        h2b = h2_ref[...]
        h3 = jnp.maximum(
            _dot_t(mean, wl_ref[...]) + _dot_t(h2b, wr_ref[...]) + bl_ref[...],
            0.0)
        w2 = w2_ref[...]
        p = (_dot_t(h1_ref[...], w2[:, 0:D]) + _dot_t(h2b, w2[:, D:2 * D])
             + _dot_t(h3, w2[:, 2 * D:3 * D]) + b2_ref[...])
        p = jnp.maximum(p, 0.0)
        o_ref[...] = _dot_t(p, w3_ref[...]) + b3_ref[...]

    D2 = 2 * D
    return pl.pallas_call(
        body,
        grid=(N // _R,),
        in_specs=[
            pl.BlockSpec((NC, _R, D), lambda i: (0, i, 0)),
            pl.BlockSpec((NC, _R, D), lambda i: (0, i, 0)),
            pl.BlockSpec((_R, D), lambda i: (i, 0)),
            pl.BlockSpec((_R, D), lambda i: (i, 0)),
            pl.BlockSpec((D, D), lambda i: (0, 0)),
            pl.BlockSpec((1, D), lambda i: (0, 0)),
            pl.BlockSpec((D, D), lambda i: (0, 0)),
            pl.BlockSpec((D2, K * D), lambda i: (0, 0)),
            pl.BlockSpec((1, D2), lambda i: (0, 0)),
            pl.BlockSpec((D, D2), lambda i: (0, 0)),
            pl.BlockSpec((1, D), lambda i: (0, 0)),
        ],
        out_specs=pl.BlockSpec((_R, D), lambda i: (i, 0)),
        out_shape=jax.ShapeDtypeStruct((N, D), jnp.float32),
    )(part, deg16, h2, h1, Wl2, bl2.reshape(1, D), Wr2,
      W2, b2.reshape(1, D2), W3, b3.reshape(1, D))


def kernel(x, edge_index, Wl0, bl0, Wr0, Wl1, bl1, Wr1, Wl2, bl2, Wr2,
           W2, b2, W3, b3):
    src = edge_index[0].astype(jnp.int32)
    dst = edge_index[1].astype(jnp.int32)
    agg = _build_agg()
    degp = _build_deg()(dst)
    part0 = agg(x, src, dst)
    h1 = _layer_tc(part0, degp, x, Wl0, bl0, Wr0)
    part1 = agg(h1, src, dst)
    h2 = _layer_tc(part1, degp, h1, Wl1, bl1, Wr1)
    part2 = agg(h2, src, dst)
    return _final_tc(part2, degp, h2, h1, Wl2, bl2, Wr2, W2, b2, W3, b3)


# speedup vs baseline: 1.0971x; 1.0005x over previous
import jax
import jax.numpy as jnp
from jax import lax
from jax.experimental import pallas as pl
from jax.experimental.pallas import tpu as pltpu
from jax.experimental.pallas import tpu_sc as plsc

N = 10000
E = 320000
D = 128
K = 3
NC = 2
NS = 16
EPC = E // NC
EPT = EPC // NS
CH = 128
NCH = EPT // CH
TAIL = EPT - NCH * CH
RPT = 624

def _mesh():
    return plsc.VectorSubcoreMesh(
        core_axis_name="c", subcore_axis_name="s",
        num_cores=NC, num_subcores=NS)


def _zero_fill(buf, nrows):
    zz = jnp.zeros((16,), jnp.float32)

    def zrow(i, carry):
        for k in range(D // 16):
            buf[i, pl.ds(k * 16, 16)] = zz
        return carry
    lax.fori_loop(0, nrows, zrow, 0)


def _acc_slices(row0):
    rem = RPT % CH
    slices = [(row0 + m * CH, CH) for m in range(RPT // CH)]
    if rem:
        slices.append((row0 + (RPT // CH) * CH, rem))
    return slices


def _zero_acc(buf, acc, row0, s, sem):
    for r0, nr in _acc_slices(row0):
        pltpu.async_copy(buf.at[pl.ds(0, nr)], acc.at[pl.ds(r0, nr)], sem)

    @pl.when(s < 2)
    def _():
        pltpu.async_copy(buf.at[pl.ds(0, 8)],
                         acc.at[pl.ds(NS * RPT + s * 8, 8)], sem)
    for r0, nr in _acc_slices(row0):
        pltpu.make_async_copy(buf.at[pl.ds(0, nr)],
                              acc.at[pl.ds(r0, nr)], sem).wait()

    @pl.when(s < 2)
    def _():
        pltpu.make_async_copy(buf.at[pl.ds(0, 8)],
                              acc.at[pl.ds(NS * RPT + s * 8, 8)], sem).wait()


def _copy_out(acc, buf, hbm_ref, row0, c, s):
    def stage_out(r0, nrows):
        pltpu.sync_copy(acc.at[pl.ds(r0, nrows)], buf.at[pl.ds(0, nrows)])
        pltpu.sync_copy(buf.at[pl.ds(0, nrows)],
                        hbm_ref.at[c, pl.ds(r0, nrows)])

    for r0, nr in _acc_slices(row0):
        stage_out(r0, nr)

    @pl.when(s < 2)
    def _():
        stage_out(NS * RPT + s * 8, 8)


def _copy_out2(acc, bufs, hbm_ref, row0, c, s, semH):
    stages = _acc_slices(row0)

    def b2h(m):
        r0, nr = stages[m]
        return pltpu.make_async_copy(bufs[m % 2].at[pl.ds(0, nr)],
                                     hbm_ref.at[c, pl.ds(r0, nr)],
                                     semH[m % 2])

    for m, (r0, nr) in enumerate(stages):
        i = m % 2
        if m >= 2:
            b2h(m - 2).wait()
        pltpu.sync_copy(acc.at[pl.ds(r0, nr)], bufs[i].at[pl.ds(0, nr)])
        b2h(m).start()
    for m in range(max(0, len(stages) - 2), len(stages)):
        b2h(m).wait()

    @pl.when(s < 2)
    def _():
        r0 = NS * RPT + s * 8
        pltpu.sync_copy(acc.at[pl.ds(r0, 8)], bufs[0].at[pl.ds(0, 8)])
        pltpu.sync_copy(bufs[0].at[pl.ds(0, 8)], hbm_ref.at[c, pl.ds(r0, 8)])


IDXW = NCH * CH
NPAIR = NCH // 2


def _stage_indices(hbm_1d, stage_1d, table_2d, base):
    pltpu.sync_copy(hbm_1d.at[pl.ds(base, IDXW)], stage_1d)

    def redis(j, carry):
        for k in range(CH // 16):
            table_2d[j, pl.ds(k * 16, 16)] = stage_1d[pl.ds(j * CH + k * 16, 16)]
        return carry
    lax.fori_loop(0, NCH, redis, 0)


def _build_agg():
    scratch = [
        pltpu.VMEM((CH,), jnp.int32),
        pltpu.VMEM((CH,), jnp.int32),
        pltpu.VMEM((CH,), jnp.int32),
        pltpu.VMEM((CH,), jnp.int32),
        pltpu.VMEM((TAIL,), jnp.int32),
        pltpu.VMEM((TAIL,), jnp.int32),
        pltpu.VMEM((CH, D), jnp.float32),
        pltpu.VMEM((CH, D), jnp.float32),
        pltpu.VMEM_SHARED((N, D), jnp.float32),
        pltpu.SemaphoreType.DMA,
        pltpu.SemaphoreType.DMA,
        pltpu.SemaphoreType.DMA,
        pltpu.SemaphoreType.DMA,
    ]

    def body(h_hbm, src_hbm, dst_hbm, out_hbm,
             srcA, dstA, srcB, dstB, srct, dstt, rowsA, rowsB, acc,
             semA, semB, semIA, semIB):
        c = lax.axis_index("c")
        s = lax.axis_index("s")
        row0 = s * RPT
        base = c * EPC + s * EPT
        _zero_fill(rowsA, CH)
        _zero_acc(rowsA, acc, row0, s, semA)
        plsc.subcore_barrier()

        def fire_idx(srcb, dstb, j, semI):
            off = base + j * CH
            pltpu.async_copy(src_hbm.at[pl.ds(off, CH)], srcb, semI)
            pltpu.async_copy(dst_hbm.at[pl.ds(off, CH)], dstb, semI)

        def wait_idx(srcb, dstb, j, semI):
            off = base + j * CH
            pltpu.make_async_copy(src_hbm.at[pl.ds(off, CH)], srcb,
                                  semI).wait()
            pltpu.make_async_copy(dst_hbm.at[pl.ds(off, CH)], dstb,
                                  semI).wait()

        fire_idx(srcA, dstA, 0, semIA)
        wait_idx(srcA, dstA, 0, semIA)
        pltpu.async_copy(h_hbm.at[srcA], rowsA, semA)
        fire_idx(srcB, dstB, 1, semIB)

        def pair(g, carry):
            a = 2 * g
            b = a + 1
            wait_idx(srcB, dstB, b, semIB)
            pltpu.async_copy(h_hbm.at[srcB], rowsB, semB)
            pltpu.make_async_copy(h_hbm.at[srcA], rowsA, semA).wait()
            pltpu.sync_copy(rowsA, acc.at[dstA], add=True)

            @pl.when(g < NPAIR - 1)
            def _():
                fire_idx(srcA, dstA, a + 2, semIA)
            pltpu.make_async_copy(h_hbm.at[srcB], rowsB, semB).wait()
            pltpu.sync_copy(rowsB, acc.at[dstB], add=True)

            @pl.when(g < NPAIR - 1)
            def _():
                wait_idx(srcA, dstA, a + 2, semIA)
                pltpu.async_copy(h_hbm.at[srcA], rowsA, semA)
                fire_idx(srcB, dstB, b + 2, semIB)
            return carry
        lax.fori_loop(0, NPAIR, pair, 0)

        pltpu.sync_copy(src_hbm.at[pl.ds(base + IDXW, TAIL)], srct)
        pltpu.sync_copy(dst_hbm.at[pl.ds(base + IDXW, TAIL)], dstt)
        pltpu.async_copy(h_hbm.at[srct], rowsA.at[pl.ds(0, TAIL)], semA).wait()
        pltpu.sync_copy(rowsA.at[pl.ds(0, TAIL)], acc.at[dstt], add=True)
        plsc.subcore_barrier()
        _copy_out2(acc, (rowsA, rowsB), out_hbm, row0, c, s,
                   (semA, semB))

    return pl.kernel(body,
                     out_type=jax.ShapeDtypeStruct((NC, N, D), jnp.float32),
                     mesh=_mesh(), scratch_types=scratch)


def _build_deg():
    scratch = [
        pltpu.VMEM((CH, D), jnp.float32),
        pltpu.VMEM((IDXW,), jnp.int32),
        pltpu.VMEM((NCH, CH), jnp.int32),
        pltpu.VMEM((TAIL,), jnp.int32),
        pltpu.VMEM_SHARED((N, D), jnp.float32),
        pltpu.SemaphoreType.DMA,
    ]

    def body(dst_hbm, out_hbm, ones, stage1, dstall, dstt, acc, sem):
        c = lax.axis_index("c")
        s = lax.axis_index("s")
        row0 = s * RPT
        base = c * EPC + s * EPT
        _stage_indices(dst_hbm, stage1, dstall, base)
        pltpu.sync_copy(dst_hbm.at[pl.ds(base + IDXW, TAIL)], dstt)
        _zero_fill(ones, CH)
        _zero_acc(ones, acc, row0, s, sem)

        oo = jnp.ones((16,), jnp.float32)

        def orow(i, carry):
            for k in range(D // 16):
                ones[i, pl.ds(k * 16, 16)] = oo
            return carry
        lax.fori_loop(0, CH, orow, 0)
        plsc.subcore_barrier()

        def fire(j, carry):
            pltpu.async_copy(ones, acc.at[dstall.at[j]], sem, add=True)
            return carry
        lax.fori_loop(0, NCH, fire, 0)

        def drain(j, carry):
            pltpu.make_async_copy(ones, acc.at[dstall.at[j]], sem).wait()
            return carry
        lax.fori_loop(0, NCH, drain, 0)
        pltpu.async_copy(ones.at[pl.ds(0, TAIL)], acc.at[dstt], sem,
                         add=True).wait()
        plsc.subcore_barrier()
        _copy_out(acc, ones, out_hbm, row0, c, s)

    return pl.kernel(body,
                     out_type=jax.ShapeDtypeStruct((NC, N, D), jnp.float32),
                     mesh=_mesh(), scratch_types=scratch)


_R = 2000


def _dot_t(a, w):
    return lax.dot_general(a, w, (((1,), (1,)), ((), ())),
                           preferred_element_type=jnp.float32)


def _mean_from_parts(part_ref, deg_ref):
    a = part_ref[0] + part_ref[1]
    dg = deg_ref[0, :, 0:1] + deg_ref[1, :, 0:1]
    return a * (1.0 / jnp.maximum(dg, 1.0))


def _layer_tc(part, deg16, h, Wl, bl, Wr):
    def body(part_ref, deg_ref, h_ref, wl_ref, bl_ref, wr_ref, o_ref):
        mean = _mean_from_parts(part_ref, deg_ref)
        acc = _dot_t(mean, wl_ref[...]) + _dot_t(h_ref[...], wr_ref[...])
        o_ref[...] = jnp.maximum(acc + bl_ref[...], 0.0)

    return pl.pallas_call(
        body,
        grid=(N // _R,),
        in_specs=[
            pl.BlockSpec((NC, _R, D), lambda i: (0, i, 0)),
            pl.BlockSpec((NC, _R, D), lambda i: (0, i, 0)),
            pl.BlockSpec((_R, D), lambda i: (i, 0)),
            pl.BlockSpec((D, D), lambda i: (0, 0)),
            pl.BlockSpec((1, D), lambda i: (0, 0)),
            pl.BlockSpec((D, D), lambda i: (0, 0)),
        ],
        out_specs=pl.BlockSpec((_R, D), lambda i: (i, 0)),
        out_shape=jax.ShapeDtypeStruct((N, D), jnp.float32),
    )(part, deg16, h, Wl, bl.reshape(1, D), Wr)


def _final_tc(part, deg16, h2, h1, Wl2, bl2, Wr2, W2, b2, W3, b3):
    def body(part_ref, deg_ref, h2_ref, h1_ref, wl_ref, bl_ref, wr_ref,
             w2_ref, b2_ref, w3_ref, b3_ref, o_ref):
        mean = _mean_from_parts(part_ref, deg_ref)
        h2b = h2_ref[...]
        h3 = jnp.maximum(
            _dot_t(mean, wl_ref[...]) + _dot_t(h2b, wr_ref[...]) + bl_ref[...],
            0.0)
        w2 = w2_ref[...]
        p = (_dot_t(h1_ref[...], w2[:, 0:D]) + _dot_t(h2b, w2[:, D:2 * D])
             + _dot_t(h3, w2[:, 2 * D:3 * D]) + b2_ref[...])
        p = jnp.maximum(p, 0.0)
        o_ref[...] = _dot_t(p, w3_ref[...]) + b3_ref[...]

    D2 = 2 * D
    return pl.pallas_call(
        body,
        grid=(N // _R,),
        in_specs=[
            pl.BlockSpec((NC, _R, D), lambda i: (0, i, 0)),
            pl.BlockSpec((NC, _R, D), lambda i: (0, i, 0)),
            pl.BlockSpec((_R, D), lambda i: (i, 0)),
            pl.BlockSpec((_R, D), lambda i: (i, 0)),
            pl.BlockSpec((D, D), lambda i: (0, 0)),
            pl.BlockSpec((1, D), lambda i: (0, 0)),
            pl.BlockSpec((D, D), lambda i: (0, 0)),
            pl.BlockSpec((D2, K * D), lambda i: (0, 0)),
            pl.BlockSpec((1, D2), lambda i: (0, 0)),
            pl.BlockSpec((D, D2), lambda i: (0, 0)),
            pl.BlockSpec((1, D), lambda i: (0, 0)),
        ],
        out_specs=pl.BlockSpec((_R, D), lambda i: (i, 0)),
        out_shape=jax.ShapeDtypeStruct((N, D), jnp.float32),
    )(part, deg16, h2, h1, Wl2, bl2.reshape(1, D), Wr2,
      W2, b2.reshape(1, D2), W3, b3.reshape(1, D))


def kernel(x, edge_index, Wl0, bl0, Wr0, Wl1, bl1, Wr1, Wl2, bl2, Wr2,
           W2, b2, W3, b3):
    src = edge_index[0].astype(jnp.int32)
    dst = edge_index[1].astype(jnp.int32)
    agg = _build_agg()
    degp = _build_deg()(dst)
    part0 = agg(x, src, dst)
    h1 = _layer_tc(part0, degp, x, Wl0, bl0, Wr0)
    part1 = agg(h1, src, dst)
    h2 = _layer_tc(part1, degp, h1, Wl1, bl1, Wr1)
    part2 = agg(h2, src, dst)
    return _final_tc(part2, degp, h2, h1, Wl2, bl2, Wr2, W2, b2, W3, b3)
